# baseline (speedup 1.0000x reference)
import jax
import jax.numpy as jnp
from jax.experimental import pallas as pl
from jax.experimental.pallas import tpu as pltpu

_K = 3
_CIN = 96
_COUT = 96
_H = 224
_W = 224
_R = 56
_LB = _R * _W
_HR = 8
_NG = _H // _R
_NH = _H // _HR - 1
_KG = _K * _CIN


def _conv_body(xp_ref, xm_ref, xn_ref, mp_ref, mm_ref, mn_ref, w_ref, b_ref,
               o_ref, xua_ref, xub_ref, xuc_ref):
    g = pl.program_id(1)
    xwin = jnp.concatenate(
        [xp_ref[0, :, _HR - 2:].astype(jnp.bfloat16),
         xm_ref[0].astype(jnp.bfloat16),
         xn_ref[0, :, :2].astype(jnp.bfloat16)], axis=1)
    xflat = xwin.reshape(_CIN, (_R + 4) * _W)
    xflat1 = jnp.concatenate(
        [xflat[:, 1:], jnp.zeros((_CIN, 1), dtype=jnp.bfloat16)], axis=1)

    lane = jax.lax.broadcasted_iota(jnp.int32, (1, _LB), 1)
    col = lane % _W
    j_first = col == 0
    j_last = col == _W - 1
    row_top = jnp.logical_and(g == 0, lane < _W)
    row_bot = jnp.logical_and(g == _NG - 1, lane >= _LB - _W)

    groups = (xua_ref, xub_ref, xuc_ref)
    acc = jnp.zeros((_COUT, _LB), dtype=jnp.float32)
    for di in range(_K):
        xu_ref = groups[di]
        for dj in range(_K):
            o = (di + 1) * _W + dj - 1
            zm = None
            if dj == 0:
                zm = j_first
            elif dj == 2:
                zm = j_last
            if di == 0:
                zm = row_top if zm is None else jnp.logical_or(zm, row_top)
            elif di == 2:
                zm = row_bot if zm is None else jnp.logical_or(zm, row_bot)
            if o % 2:
                xs = xflat1[:, o - 1:o - 1 + _LB]
            else:
                xs = xflat[:, o:o + _LB]
            if zm is not None:
                xs = jnp.where(zm, jnp.bfloat16(0), xs)
            xu_ref[dj * _CIN:(dj + 1) * _CIN, :] = xs
        acc += jnp.dot(w_ref[:, di * _KG:(di + 1) * _KG], xu_ref[...],
                       preferred_element_type=jnp.float32)

    m3 = jnp.concatenate(
        [mp_ref[0, 0, _HR - 2:], mm_ref[0, 0], mn_ref[0, 0, :2]], axis=0)
    zc = jnp.zeros((_R + 4, 1), dtype=jnp.float32)
    m3p = jnp.concatenate([zc, m3, zc], axis=1)
    rows = jax.lax.broadcasted_iota(jnp.int32, (_R, _W), 0)
    rt2 = jnp.logical_and(g == 0, rows < 1)
    rb2 = jnp.logical_and(g == _NG - 1, rows >= _R - 1)
    mwin = jnp.zeros((_R, _W), dtype=jnp.float32)
    for di in range(_K):
        for dj in range(_K):
            sl = m3p[1 + di:1 + di + _R, dj:dj + _W]
            if di == 0:
                sl = jnp.where(rt2, 0.0, sl)
            elif di == 2:
                sl = jnp.where(rb2, 0.0, sl)
            mwin = jnp.maximum(mwin, jnp.abs(sl))

    out = (acc + b_ref[...]).reshape(_COUT, _R, _W)
    o_ref[0] = jnp.where(mwin[None] != 0, out, 0.0)


def kernel(x, mask, weight, bias):
    b = x.shape[0]
    w2 = jnp.transpose(weight, (0, 2, 3, 1)).reshape(_COUT, _K * _K * _CIN)
    w2 = w2.astype(jnp.bfloat16)
    b2 = bias.reshape(_COUT, 1)

    nh = _R // _HR

    grid = (b, _NG)
    out = pl.pallas_call(
        _conv_body,
        grid=grid,
        in_specs=[
            pl.BlockSpec((1, _CIN, _HR, _W),
                         lambda bb, g: (bb, 0, jnp.maximum(g * nh - 1, 0), 0)),
            pl.BlockSpec((1, _CIN, _R, _W), lambda bb, g: (bb, 0, g, 0)),
            pl.BlockSpec((1, _CIN, _HR, _W),
                         lambda bb, g: (bb, 0,
                                        jnp.minimum((g + 1) * nh, _NH), 0)),
            pl.BlockSpec((1, 1, _HR, _W),
                         lambda bb, g: (bb, 0, jnp.maximum(g * nh - 1, 0), 0)),
            pl.BlockSpec((1, 1, _R, _W), lambda bb, g: (bb, 0, g, 0)),
            pl.BlockSpec((1, 1, _HR, _W),
                         lambda bb, g: (bb, 0,
                                        jnp.minimum((g + 1) * nh, _NH), 0)),
            pl.BlockSpec((_COUT, _K * _K * _CIN), lambda bb, g: (0, 0)),
            pl.BlockSpec((_COUT, 1), lambda bb, g: (0, 0)),
        ],
        out_specs=pl.BlockSpec((1, _COUT, _R, _W), lambda bb, g: (bb, 0, g, 0)),
        out_shape=jax.ShapeDtypeStruct((b, _COUT, _H, _W), jnp.float32),
        scratch_shapes=[pltpu.VMEM((_KG, _LB), jnp.bfloat16),
                        pltpu.VMEM((_KG, _LB), jnp.bfloat16),
                        pltpu.VMEM((_KG, _LB), jnp.bfloat16)],
    )(x, x, x, mask, mask, mask, w2, b2)
    return out


# speedup vs baseline: 1.0455x; 1.0455x over previous
import jax
import jax.numpy as jnp
from jax.experimental import pallas as pl
from jax.experimental.pallas import tpu as pltpu

_K = 3
_CIN = 96
_COUT = 96
_H = 224
_W = 224
_R = 56
_LB = _R * _W
_HR = 8
_NG = _H // _R
_NH = _H // _HR - 1
_KG = _K * _CIN


def _conv_body(xp_ref, xm_ref, xn_ref, mp_ref, mm_ref, mn_ref, w_ref, b_ref,
               o_ref, xua_ref, xub_ref, xuc_ref):
    g = pl.program_id(1)
    xwin = jnp.concatenate(
        [xp_ref[0, :, _HR - 2:].astype(jnp.bfloat16),
         xm_ref[0].astype(jnp.bfloat16),
         xn_ref[0, :, :2].astype(jnp.bfloat16)], axis=1)
    xflat = xwin.reshape(_CIN, (_R + 4) * _W)

    lane = jax.lax.broadcasted_iota(jnp.int32, (1, _LB), 1)
    col = lane % _W
    j_first = col == 0
    j_last = col == _W - 1
    row_top = jnp.logical_and(g == 0, lane < _W)
    row_bot = jnp.logical_and(g == _NG - 1, lane >= _LB - _W)

    groups = (xua_ref, xub_ref, xuc_ref)
    acc = jnp.zeros((_COUT, _LB), dtype=jnp.float32)
    for di in range(_K):
        xu_ref = groups[di]
        for dj in range(_K):
            o = (di + 1) * _W + dj - 1
            zm = None
            if dj == 0:
                zm = j_first
            elif dj == 2:
                zm = j_last
            if di == 0:
                zm = row_top if zm is None else jnp.logical_or(zm, row_top)
            elif di == 2:
                zm = row_bot if zm is None else jnp.logical_or(zm, row_bot)
            xs = xflat[:, o:o + _LB]
            if zm is not None:
                xs = jnp.where(zm, jnp.bfloat16(0), xs)
            xu_ref[dj * _CIN:(dj + 1) * _CIN, :] = xs
        acc += jnp.dot(w_ref[:, di * _KG:(di + 1) * _KG], xu_ref[...],
                       preferred_element_type=jnp.float32)

    m3 = jnp.concatenate(
        [mp_ref[0, 0, _HR - 2:], mm_ref[0, 0], mn_ref[0, 0, :2]], axis=0)
    zc = jnp.zeros((_R + 4, 1), dtype=jnp.float32)
    m3p = jnp.concatenate([zc, m3, zc], axis=1)
    rows = jax.lax.broadcasted_iota(jnp.int32, (_R, _W), 0)
    rt2 = jnp.logical_and(g == 0, rows < 1)
    rb2 = jnp.logical_and(g == _NG - 1, rows >= _R - 1)
    mwin = jnp.zeros((_R, _W), dtype=jnp.float32)
    for di in range(_K):
        for dj in range(_K):
            sl = m3p[1 + di:1 + di + _R, dj:dj + _W]
            if di == 0:
                sl = jnp.where(rt2, 0.0, sl)
            elif di == 2:
                sl = jnp.where(rb2, 0.0, sl)
            mwin = jnp.maximum(mwin, jnp.abs(sl))

    out = (acc + b_ref[...]).reshape(_COUT, _R, _W)
    o_ref[0] = jnp.where(mwin[None] != 0, out, 0.0)


def kernel(x, mask, weight, bias):
    b = x.shape[0]
    w2 = jnp.transpose(weight, (0, 2, 3, 1)).reshape(_COUT, _K * _K * _CIN)
    w2 = w2.astype(jnp.bfloat16)
    b2 = bias.reshape(_COUT, 1)

    nh = _R // _HR

    grid = (b, _NG)
    out = pl.pallas_call(
        _conv_body,
        grid=grid,
        in_specs=[
            pl.BlockSpec((1, _CIN, _HR, _W),
                         lambda bb, g: (bb, 0, jnp.maximum(g * nh - 1, 0), 0)),
            pl.BlockSpec((1, _CIN, _R, _W), lambda bb, g: (bb, 0, g, 0)),
            pl.BlockSpec((1, _CIN, _HR, _W),
                         lambda bb, g: (bb, 0,
                                        jnp.minimum((g + 1) * nh, _NH), 0)),
            pl.BlockSpec((1, 1, _HR, _W),
                         lambda bb, g: (bb, 0, jnp.maximum(g * nh - 1, 0), 0)),
            pl.BlockSpec((1, 1, _R, _W), lambda bb, g: (bb, 0, g, 0)),
            pl.BlockSpec((1, 1, _HR, _W),
                         lambda bb, g: (bb, 0,
                                        jnp.minimum((g + 1) * nh, _NH), 0)),
            pl.BlockSpec((_COUT, _K * _K * _CIN), lambda bb, g: (0, 0)),
            pl.BlockSpec((_COUT, 1), lambda bb, g: (0, 0)),
        ],
        out_specs=pl.BlockSpec((1, _COUT, _R, _W), lambda bb, g: (bb, 0, g, 0)),
        out_shape=jax.ShapeDtypeStruct((b, _COUT, _H, _W), jnp.float32),
        scratch_shapes=[pltpu.VMEM((_KG, _LB), jnp.bfloat16),
                        pltpu.VMEM((_KG, _LB), jnp.bfloat16),
                        pltpu.VMEM((_KG, _LB), jnp.bfloat16)],
    )(x, x, x, mask, mask, mask, w2, b2)
    return out


# speedup vs baseline: 1.0479x; 1.0023x over previous
import jax
import jax.numpy as jnp
from jax.experimental import pallas as pl
from jax.experimental.pallas import tpu as pltpu

_K = 3
_CIN = 96
_COUT = 96
_H = 224
_W = 224
_R = 56
_LB = _R * _W
_HR = 8
_NG = _H // _R
_NH = _H // _HR - 1
_KG = _K * _CIN


def _conv_body(xp_ref, xm_ref, xn_ref, mp_ref, mm_ref, mn_ref, w_ref, b_ref,
               o_ref, xua_ref, xub_ref, xuc_ref):
    g = pl.program_id(1)
    xwin = jnp.concatenate(
        [xp_ref[0, :, _HR - 2:].astype(jnp.bfloat16),
         xm_ref[0].astype(jnp.bfloat16),
         xn_ref[0, :, :2].astype(jnp.bfloat16)], axis=1)
    xflat = xwin.reshape(_CIN, (_R + 4) * _W)

    lane = jax.lax.broadcasted_iota(jnp.int32, (1, _LB), 1)
    col = lane % _W
    j_first = col == 0
    j_last = col == _W - 1
    row_top = jnp.logical_and(g == 0, lane < _W)
    row_bot = jnp.logical_and(g == _NG - 1, lane >= _LB - _W)

    groups = (xua_ref, xub_ref, xuc_ref)
    acc = jnp.zeros((_COUT, _LB), dtype=jnp.float32)
    for di in range(_K):
        xu_ref = groups[di]
        for dj in range(_K):
            o = (di + 1) * _W + dj - 1
            zm = None
            if dj == 0:
                zm = j_first
            elif dj == 2:
                zm = j_last
            if di == 0:
                zm = row_top if zm is None else jnp.logical_or(zm, row_top)
            elif di == 2:
                zm = row_bot if zm is None else jnp.logical_or(zm, row_bot)
            xs = xflat[:, o:o + _LB]
            if zm is not None:
                xs = jnp.where(zm, jnp.bfloat16(0), xs)
            xu_ref[dj * _CIN:(dj + 1) * _CIN, :] = xs
        acc += jnp.dot(w_ref[:, di * _KG:(di + 1) * _KG], xu_ref[...],
                       preferred_element_type=jnp.float32)

    m3 = jnp.concatenate(
        [mp_ref[0, 0, _HR - 2:], mm_ref[0, 0], mn_ref[0, 0, :2]], axis=0)
    zc = jnp.zeros((_R + 4, 1), dtype=jnp.float32)
    m3p = jnp.concatenate([zc, m3, zc], axis=1)
    rows = jax.lax.broadcasted_iota(jnp.int32, (_R, _W), 0)
    rt2 = jnp.logical_and(g == 0, rows < 1)
    rb2 = jnp.logical_and(g == _NG - 1, rows >= _R - 1)
    mwin = jnp.zeros((_R, _W), dtype=jnp.float32)
    for di in range(_K):
        for dj in range(_K):
            sl = m3p[1 + di:1 + di + _R, dj:dj + _W]
            if di == 0:
                sl = jnp.where(rt2, 0.0, sl)
            elif di == 2:
                sl = jnp.where(rb2, 0.0, sl)
            mwin = jnp.maximum(mwin, jnp.abs(sl))

    out = (acc + b_ref[...]).reshape(_COUT, _R, _W)
    o_ref[0] = jnp.where(mwin[None] != 0, out, 0.0)


def kernel(x, mask, weight, bias):
    b = x.shape[0]
    w2 = jnp.transpose(weight, (0, 2, 3, 1)).reshape(_COUT, _K * _K * _CIN)
    w2 = w2.astype(jnp.bfloat16)
    b2 = bias.reshape(_COUT, 1)

    nh = _R // _HR

    grid = (b, _NG)
    out = pl.pallas_call(
        _conv_body,
        grid=grid,
        in_specs=[
            pl.BlockSpec((1, _CIN, _HR, _W),
                         lambda bb, g: (bb, 0, jnp.maximum(g * nh - 1, 0), 0)),
            pl.BlockSpec((1, _CIN, _R, _W), lambda bb, g: (bb, 0, g, 0)),
            pl.BlockSpec((1, _CIN, _HR, _W),
                         lambda bb, g: (bb, 0,
                                        jnp.minimum((g + 1) * nh, _NH), 0)),
            pl.BlockSpec((1, 1, _HR, _W),
                         lambda bb, g: (bb, 0, jnp.maximum(g * nh - 1, 0), 0)),
            pl.BlockSpec((1, 1, _R, _W), lambda bb, g: (bb, 0, g, 0)),
            pl.BlockSpec((1, 1, _HR, _W),
                         lambda bb, g: (bb, 0,
                                        jnp.minimum((g + 1) * nh, _NH), 0)),
            pl.BlockSpec((_COUT, _K * _K * _CIN), lambda bb, g: (0, 0)),
            pl.BlockSpec((_COUT, 1), lambda bb, g: (0, 0)),
        ],
        out_specs=pl.BlockSpec((1, _COUT, _R, _W), lambda bb, g: (bb, 0, g, 0)),
        out_shape=jax.ShapeDtypeStruct((b, _COUT, _H, _W), jnp.float32),
        scratch_shapes=[pltpu.VMEM((_KG, _LB), jnp.bfloat16),
                        pltpu.VMEM((_KG, _LB), jnp.bfloat16),
                        pltpu.VMEM((_KG, _LB), jnp.bfloat16)],
        compiler_params=pltpu.CompilerParams(
            dimension_semantics=("parallel", "arbitrary")),
    )(x, x, x, mask, mask, mask, w2, b2)
    return out


# speedup vs baseline: 1.0488x; 1.0008x over previous
import jax
import jax.numpy as jnp
from jax.experimental import pallas as pl
from jax.experimental.pallas import tpu as pltpu

_K = 3
_CIN = 96
_COUT = 96
_H = 224
_W = 224
_R = 56
_LB = _R * _W
_HR = 8
_NG = _H // _R
_NH = _H // _HR - 1
_KG = _K * _CIN


def _conv_body(xp_ref, xm_ref, xn_ref, mp_ref, mm_ref, mn_ref, w_ref, b_ref,
               o_ref, xua_ref, xub_ref, xuc_ref):
    g = pl.program_id(1)
    xwin = jnp.concatenate(
        [xp_ref[0, :, _HR - 2:].astype(jnp.bfloat16),
         xm_ref[0].astype(jnp.bfloat16),
         xn_ref[0, :, :2].astype(jnp.bfloat16)], axis=1)
    xflat = xwin.reshape(_CIN, (_R + 4) * _W)

    lane = jax.lax.broadcasted_iota(jnp.int32, (1, _LB), 1)
    col = lane % _W
    j_first = col == 0
    j_last = col == _W - 1
    row_top = jnp.logical_and(g == 0, lane < _W)
    row_bot = jnp.logical_and(g == _NG - 1, lane >= _LB - _W)

    groups = (xua_ref, xub_ref, xuc_ref)
    for di in range(_K):
        xu_ref = groups[di]
        for dj in range(_K):
            o = (di + 1) * _W + dj - 1
            zm = None
            if dj == 0:
                zm = j_first
            elif dj == 2:
                zm = j_last
            if di == 0:
                zm = row_top if zm is None else jnp.logical_or(zm, row_top)
            elif di == 2:
                zm = row_bot if zm is None else jnp.logical_or(zm, row_bot)
            xs = xflat[:, o:o + _LB]
            if zm is not None:
                xs = jnp.where(zm, jnp.bfloat16(0), xs)
            xu_ref[dj * _CIN:(dj + 1) * _CIN, :] = xs
    acc = jnp.dot(w_ref[:, :_KG], xua_ref[...],
                  preferred_element_type=jnp.float32)
    acc += jnp.dot(w_ref[:, _KG:2 * _KG], xub_ref[...],
                   preferred_element_type=jnp.float32)
    acc += jnp.dot(w_ref[:, 2 * _KG:], xuc_ref[...],
                   preferred_element_type=jnp.float32)

    m3 = jnp.concatenate(
        [mp_ref[0, 0, _HR - 2:], mm_ref[0, 0], mn_ref[0, 0, :2]], axis=0)
    zc = jnp.zeros((_R + 4, 1), dtype=jnp.float32)
    m3p = jnp.concatenate([zc, m3, zc], axis=1)
    rows = jax.lax.broadcasted_iota(jnp.int32, (_R, _W), 0)
    rt2 = jnp.logical_and(g == 0, rows < 1)
    rb2 = jnp.logical_and(g == _NG - 1, rows >= _R - 1)
    mwin = jnp.zeros((_R, _W), dtype=jnp.float32)
    for di in range(_K):
        for dj in range(_K):
            sl = m3p[1 + di:1 + di + _R, dj:dj + _W]
            if di == 0:
                sl = jnp.where(rt2, 0.0, sl)
            elif di == 2:
                sl = jnp.where(rb2, 0.0, sl)
            mwin = jnp.maximum(mwin, jnp.abs(sl))

    out = (acc + b_ref[...]).reshape(_COUT, _R, _W)
    o_ref[0] = jnp.where(mwin[None] != 0, out, 0.0)


def kernel(x, mask, weight, bias):
    b = x.shape[0]
    w2 = jnp.transpose(weight, (0, 2, 3, 1)).reshape(_COUT, _K * _K * _CIN)
    w2 = w2.astype(jnp.bfloat16)
    b2 = bias.reshape(_COUT, 1)

    nh = _R // _HR

    grid = (b, _NG)
    out = pl.pallas_call(
        _conv_body,
        grid=grid,
        in_specs=[
            pl.BlockSpec((1, _CIN, _HR, _W),
                         lambda bb, g: (bb, 0, jnp.maximum(g * nh - 1, 0), 0)),
            pl.BlockSpec((1, _CIN, _R, _W), lambda bb, g: (bb, 0, g, 0)),
            pl.BlockSpec((1, _CIN, _HR, _W),
                         lambda bb, g: (bb, 0,
                                        jnp.minimum((g + 1) * nh, _NH), 0)),
            pl.BlockSpec((1, 1, _HR, _W),
                         lambda bb, g: (bb, 0, jnp.maximum(g * nh - 1, 0), 0)),
            pl.BlockSpec((1, 1, _R, _W), lambda bb, g: (bb, 0, g, 0)),
            pl.BlockSpec((1, 1, _HR, _W),
                         lambda bb, g: (bb, 0,
                                        jnp.minimum((g + 1) * nh, _NH), 0)),
            pl.BlockSpec((_COUT, _K * _K * _CIN), lambda bb, g: (0, 0)),
            pl.BlockSpec((_COUT, 1), lambda bb, g: (0, 0)),
        ],
        out_specs=pl.BlockSpec((1, _COUT, _R, _W), lambda bb, g: (bb, 0, g, 0)),
        out_shape=jax.ShapeDtypeStruct((b, _COUT, _H, _W), jnp.float32),
        scratch_shapes=[pltpu.VMEM((_KG, _LB), jnp.bfloat16),
                        pltpu.VMEM((_KG, _LB), jnp.bfloat16),
                        pltpu.VMEM((_KG, _LB), jnp.bfloat16)],
        compiler_params=pltpu.CompilerParams(
            dimension_semantics=("parallel", "arbitrary")),
    )(x, x, x, mask, mask, mask, w2, b2)
    return out
